# baseline (speedup 1.0000x reference)
import functools

import jax
import jax.numpy as jnp
from jax import lax
from jax.experimental import pallas as pl
from jax.experimental.pallas import tpu as pltpu
from jax.experimental.pallas import tpu_sc as plsc

_K0 = 2547012911
_K1 = 1371500959
_KS2 = (_K0 ^ _K1 ^ 0x1BD11BDA) & 0xFFFFFFFF
_ROT_A = (13, 15, 26, 6)
_ROT_B = (17, 29, 16, 24)
_KS = (_K0, _K1, _KS2)
_INJ = tuple(
    (_KS[(i + 1) % 3], (_KS[(i + 2) % 3] + i + 1) & 0xFFFFFFFF) for i in range(5)
)

_NW = 32
_W_PER = 7168
_SC_COLS = _NW * _W_PER
_RG = 8
_Q_COEF = (
    1.4242258145027955, -1.676855335983351, 1.2559217275518721,
    -0.2095911090593185, 0.2730018212528344, 0.242037642372466,
    0.3337806931142869, 0.49998943875471275, 1.000000060512383,
)


def _threefry_bits(cnt):
    x0 = jnp.full_like(cnt, jnp.uint32(_K0))
    x1 = cnt + jnp.uint32(_K1)
    for i, rots in enumerate((_ROT_A, _ROT_B, _ROT_A, _ROT_B, _ROT_A)):
        for r in rots:
            x0 = x0 + x1
            x1 = (x1 << jnp.uint32(r)) | (x1 >> jnp.uint32(32 - r))
            x1 = x1 ^ x0
        ca, cb = _INJ[i]
        x0 = x0 + jnp.uint32(ca)
        x1 = x1 + jnp.uint32(cb)
    return x0 ^ x1


def _bits_to_u(bits):
    fbits = (bits >> jnp.uint32(9)) | jnp.uint32(0x3F800000)
    return lax.bitcast_convert_type(fbits, jnp.float32) - 1.0


def _tc_score(x, cnt):
    u = _bits_to_u(_threefry_bits(cnt))
    w = -jnp.log(u + 1e-20)
    return jnp.log(jnp.maximum(x, 1e-30)) - jnp.log(w)


def _tc_body(x_ref, o_ref, acc_ref, idx_ref, *, stride, limit, col_off, blk, ch):
    pid = pl.program_id(0)
    nprog = pl.num_programs(0)
    rows = x_ref.shape[0]
    nch = blk // ch

    @pl.when(pid == 0)
    def _init():
        acc_ref[...] = jnp.full(acc_ref.shape, -jnp.inf, acc_ref.dtype)
        idx_ref[...] = jnp.zeros(idx_ref.shape, idx_ref.dtype)

    row = lax.broadcasted_iota(jnp.uint32, (rows, ch), 0)
    colv = lax.broadcasted_iota(jnp.uint32, (rows, ch), 1)
    base_vec = row * jnp.uint32(stride) + colv + jnp.uint32(col_off)
    lane = lax.broadcasted_iota(jnp.int32, (rows, ch), 1)

    def chunk(j, carry, masked):
        acc, idxa = carry
        jf = pid * nch + j
        x = x_ref[:, pl.ds(pl.multiple_of(j * ch, ch), ch)]
        cnt = base_vec + (jf * ch).astype(jnp.uint32)
        s = _tc_score(x, cnt)
        if masked:
            cidx = lane + (col_off + jf * ch)
            s = jnp.where(cidx < limit, s, -jnp.inf)
        better = s > acc
        acc = jnp.where(better, s, acc)
        idxa = jnp.where(better, jf, idxa)
        return acc, idxa

    carry0 = (acc_ref[...], idx_ref[...])

    @pl.when(pid != nprog - 1)
    def _main():
        acc, idxa = lax.fori_loop(
            0, nch, functools.partial(chunk, masked=False), carry0, unroll=4
        )
        acc_ref[...] = acc
        idx_ref[...] = idxa

    @pl.when(pid == nprog - 1)
    def _tail():
        acc, idxa = lax.fori_loop(
            0, nch, functools.partial(chunk, masked=True), carry0, unroll=4
        )
        m = jnp.max(acc, axis=1, keepdims=True)
        gidx = idxa * ch + lane + col_off
        o_ref[...] = jnp.min(
            jnp.where(acc == m, gidx, jnp.int32(2**31 - 1)), axis=1, keepdims=True
        )


def _tc_argmax(x, col_off):
    rows, ncols = x.shape
    ch = 128
    blk = 8192
    assert col_off % blk == 0
    n = pl.cdiv(ncols - col_off, blk)
    off_blocks = col_off // blk
    return pl.pallas_call(
        functools.partial(
            _tc_body, stride=ncols, limit=ncols, col_off=col_off, blk=blk, ch=ch
        ),
        grid=(n,),
        in_specs=[pl.BlockSpec((rows, blk), lambda i: (0, i + off_blocks))],
        out_specs=pl.BlockSpec((rows, 1), lambda i: (0, 0)),
        out_shape=jax.ShapeDtypeStruct((rows, 1), jnp.int32),
        scratch_shapes=[
            pltpu.VMEM((rows, ch), jnp.float32),
            pltpu.VMEM((rows, ch), jnp.int32),
        ],
    )(x)


def _neglog_rel(u):
    d = 1.0 - u
    q = jnp.full_like(u, _Q_COEF[0])
    for c in _Q_COEF[1:]:
        q = q * d + c
    return jnp.where(u >= 0.5, d * q, 46.1)


def _sc_shard_winners(x, rows, ncols):
    mesh = plsc.VectorSubcoreMesh(core_axis_name="c", subcore_axis_name="s")

    @functools.partial(
        pl.kernel,
        mesh=mesh,
        out_type=jax.ShapeDtypeStruct((_NW, rows, 16), jnp.int32),
        scratch_types=[
            pltpu.VMEM((_RG, _W_PER), jnp.float32),
            pltpu.VMEM((rows, 16), jnp.int32),
        ],
    )
    def k(x_hbm, oi_hbm, xbuf, idxv):
        wid = lax.axis_index("s") * 2 + lax.axis_index("c")
        base_col = wid * _W_PER
        lane = lax.iota(jnp.int32, 16)

        def rg_body(r8, _):
            r0 = r8 * _RG
            pltpu.sync_copy(
                x_hbm.at[pl.ds(r0, _RG), pl.ds(base_col, _W_PER)], xbuf
            )
            for rr in range(_RG):
                r = r0 + rr
                base_cnt = r * ncols + base_col

                def vec_body(v, carry, rr=rr, base_cnt=base_cnt):
                    xa, wa, idx = carry
                    xv = xbuf[rr, pl.ds(v * 16, 16)]
                    cnt = (lane + (base_cnt + v * 16)).astype(jnp.uint32)
                    u = _bits_to_u(_threefry_bits(cnt))
                    w = _neglog_rel(u)
                    xv = jnp.maximum(xv, 1e-30)
                    better = xv * wa > xa * w
                    xa = jnp.where(better, xv, xa)
                    wa = jnp.where(better, w, wa)
                    idx = jnp.where(better, v, idx)
                    return xa, wa, idx

                _, _, idx = lax.fori_loop(
                    0,
                    _W_PER // 16,
                    vec_body,
                    (
                        jnp.zeros((16,), jnp.float32),
                        jnp.ones((16,), jnp.float32),
                        jnp.zeros((16,), jnp.int32),
                    ),
                )
                idxv[r] = idx
            return 0

        lax.fori_loop(0, rows // _RG, rg_body, 0)
        pltpu.sync_copy(idxv, oi_hbm.at[wid])

    return k(x)


@jax.jit
def kernel(x):
    rows, ncols = x.shape

    tc_idx = _tc_argmax(x, _SC_COLS)
    sc_vec = _sc_shard_winners(x, rows, ncols)

    widc = jnp.arange(_NW, dtype=jnp.int32)[:, None, None]
    lanec = jnp.arange(16, dtype=jnp.int32)[None, None, :]
    cand = (widc * _W_PER + sc_vec * 16 + lanec).transpose(1, 0, 2)
    cols = jnp.concatenate(
        [tc_idx, cand.reshape(rows, _NW * 16)], axis=1
    )

    xg = jnp.take_along_axis(x, cols, axis=1)
    cnt = (jnp.arange(rows, dtype=jnp.int32)[:, None] * ncols + cols).astype(
        jnp.uint32
    )
    u = _bits_to_u(_threefry_bits(cnt))
    g = -jnp.log(-jnp.log(u + 1e-20) + 1e-20)
    s = jnp.log(jnp.clip(xg, 1e-30, None)) + g
    m = jnp.max(s, axis=1, keepdims=True)
    win = jnp.min(
        jnp.where(s == m, cols, jnp.int32(2**31 - 1)), axis=1, keepdims=True
    )
    return win


# speedup vs baseline: 1.0029x; 1.0029x over previous
import functools

import jax
import jax.numpy as jnp
from jax import lax
from jax.experimental import pallas as pl
from jax.experimental.pallas import tpu as pltpu
from jax.experimental.pallas import tpu_sc as plsc

_K0 = 2547012911
_K1 = 1371500959
_KS2 = (_K0 ^ _K1 ^ 0x1BD11BDA) & 0xFFFFFFFF
_ROT_A = (13, 15, 26, 6)
_ROT_B = (17, 29, 16, 24)
_KS = (_K0, _K1, _KS2)
_INJ = tuple(
    (_KS[(i + 1) % 3], (_KS[(i + 2) % 3] + i + 1) & 0xFFFFFFFF) for i in range(5)
)

_NW = 32
_W_PER = 7424
_SC_COLS = _NW * _W_PER
_RG = 8
_Q_COEF = (
    1.4242258145027955, -1.676855335983351, 1.2559217275518721,
    -0.2095911090593185, 0.2730018212528344, 0.242037642372466,
    0.3337806931142869, 0.49998943875471275, 1.000000060512383,
)


def _threefry_bits(cnt):
    x0 = jnp.full_like(cnt, jnp.uint32(_K0))
    x1 = cnt + jnp.uint32(_K1)
    for i, rots in enumerate((_ROT_A, _ROT_B, _ROT_A, _ROT_B, _ROT_A)):
        for r in rots:
            x0 = x0 + x1
            x1 = (x1 << jnp.uint32(r)) | (x1 >> jnp.uint32(32 - r))
            x1 = x1 ^ x0
        ca, cb = _INJ[i]
        x0 = x0 + jnp.uint32(ca)
        x1 = x1 + jnp.uint32(cb)
    return x0 ^ x1


def _bits_to_u(bits):
    fbits = (bits >> jnp.uint32(9)) | jnp.uint32(0x3F800000)
    return lax.bitcast_convert_type(fbits, jnp.float32) - 1.0


def _tc_score(x, cnt):
    u = _bits_to_u(_threefry_bits(cnt))
    w = -jnp.log(u + 1e-20)
    return jnp.log(jnp.maximum(x, 1e-30)) - jnp.log(w)


def _tc_body(x_ref, o_ref, acc_ref, idx_ref, *, stride, limit, col_off, blk, ch):
    pid = pl.program_id(0)
    nprog = pl.num_programs(0)
    rows = x_ref.shape[0]
    nch = blk // ch

    @pl.when(pid == 0)
    def _init():
        acc_ref[...] = jnp.full(acc_ref.shape, -jnp.inf, acc_ref.dtype)
        idx_ref[...] = jnp.zeros(idx_ref.shape, idx_ref.dtype)

    row = lax.broadcasted_iota(jnp.uint32, (rows, ch), 0)
    colv = lax.broadcasted_iota(jnp.uint32, (rows, ch), 1)
    base_vec = row * jnp.uint32(stride) + colv + jnp.uint32(col_off)
    lane = lax.broadcasted_iota(jnp.int32, (rows, ch), 1)

    def chunk(j, carry, masked):
        acc, idxa = carry
        jf = pid * nch + j
        x = x_ref[:, pl.ds(pl.multiple_of(j * ch, ch), ch)]
        cnt = base_vec + (jf * ch).astype(jnp.uint32)
        s = _tc_score(x, cnt)
        if masked:
            cidx = lane + (col_off + jf * ch)
            s = jnp.where(cidx < limit, s, -jnp.inf)
        better = s > acc
        acc = jnp.where(better, s, acc)
        idxa = jnp.where(better, jf, idxa)
        return acc, idxa

    carry0 = (acc_ref[...], idx_ref[...])

    @pl.when(pid != nprog - 1)
    def _main():
        acc, idxa = lax.fori_loop(
            0, nch, functools.partial(chunk, masked=False), carry0, unroll=4
        )
        acc_ref[...] = acc
        idx_ref[...] = idxa

    @pl.when(pid == nprog - 1)
    def _tail():
        acc, idxa = lax.fori_loop(
            0, nch, functools.partial(chunk, masked=True), carry0, unroll=4
        )
        m = jnp.max(acc, axis=1, keepdims=True)
        gidx = idxa * ch + lane + col_off
        o_ref[...] = jnp.min(
            jnp.where(acc == m, gidx, jnp.int32(2**31 - 1)), axis=1, keepdims=True
        )


def _tc_argmax(x, col_off):
    rows, ncols = x.shape
    ch = 128
    blk = 8192
    assert col_off % blk == 0
    n = pl.cdiv(ncols - col_off, blk)
    off_blocks = col_off // blk
    return pl.pallas_call(
        functools.partial(
            _tc_body, stride=ncols, limit=ncols, col_off=col_off, blk=blk, ch=ch
        ),
        grid=(n,),
        in_specs=[pl.BlockSpec((rows, blk), lambda i: (0, i + off_blocks))],
        out_specs=pl.BlockSpec((rows, 1), lambda i: (0, 0)),
        out_shape=jax.ShapeDtypeStruct((rows, 1), jnp.int32),
        scratch_shapes=[
            pltpu.VMEM((rows, ch), jnp.float32),
            pltpu.VMEM((rows, ch), jnp.int32),
        ],
    )(x)


def _neglog_rel(u):
    d = 1.0 - u
    q = jnp.full_like(u, _Q_COEF[0])
    for c in _Q_COEF[1:]:
        q = q * d + c
    return jnp.where(u >= 0.5, d * q, 46.1)


def _sc_shard_winners(x, rows, ncols):
    mesh = plsc.VectorSubcoreMesh(core_axis_name="c", subcore_axis_name="s")

    @functools.partial(
        pl.kernel,
        mesh=mesh,
        out_type=jax.ShapeDtypeStruct((_NW, rows, 16), jnp.int32),
        scratch_types=[
            pltpu.VMEM((_RG, _W_PER), jnp.float32),
            pltpu.VMEM((rows, 16), jnp.int32),
        ],
    )
    def k(x_hbm, oi_hbm, xbuf, idxv):
        wid = lax.axis_index("s") * 2 + lax.axis_index("c")
        base_col = wid * _W_PER
        lane = lax.iota(jnp.int32, 16)

        def rg_body(r8, _):
            r0 = r8 * _RG
            pltpu.sync_copy(
                x_hbm.at[pl.ds(r0, _RG), pl.ds(base_col, _W_PER)], xbuf
            )
            for rr in range(_RG):
                r = r0 + rr
                base_cnt = r * ncols + base_col

                def vec_body(v, carry, rr=rr, base_cnt=base_cnt):
                    xa, wa, idx = carry
                    xv = xbuf[rr, pl.ds(v * 16, 16)]
                    cnt = (lane + (base_cnt + v * 16)).astype(jnp.uint32)
                    u = _bits_to_u(_threefry_bits(cnt))
                    w = _neglog_rel(u)
                    xv = jnp.maximum(xv, 1e-30)
                    better = xv * wa > xa * w
                    xa = jnp.where(better, xv, xa)
                    wa = jnp.where(better, w, wa)
                    idx = jnp.where(better, v, idx)
                    return xa, wa, idx

                _, _, idx = lax.fori_loop(
                    0,
                    _W_PER // 16,
                    vec_body,
                    (
                        jnp.zeros((16,), jnp.float32),
                        jnp.ones((16,), jnp.float32),
                        jnp.zeros((16,), jnp.int32),
                    ),
                )
                idxv[r] = idx
            return 0

        lax.fori_loop(0, rows // _RG, rg_body, 0)
        pltpu.sync_copy(idxv, oi_hbm.at[wid])

    return k(x)


@jax.jit
def kernel(x):
    rows, ncols = x.shape

    tc_idx = _tc_argmax(x, _SC_COLS)
    sc_vec = _sc_shard_winners(x, rows, ncols)

    widc = jnp.arange(_NW, dtype=jnp.int32)[:, None, None]
    lanec = jnp.arange(16, dtype=jnp.int32)[None, None, :]
    cand = (widc * _W_PER + sc_vec * 16 + lanec).transpose(1, 0, 2)
    cols = jnp.concatenate(
        [tc_idx, cand.reshape(rows, _NW * 16)], axis=1
    )

    xg = jnp.take_along_axis(x, cols, axis=1)
    cnt = (jnp.arange(rows, dtype=jnp.int32)[:, None] * ncols + cols).astype(
        jnp.uint32
    )
    u = _bits_to_u(_threefry_bits(cnt))
    g = -jnp.log(-jnp.log(u + 1e-20) + 1e-20)
    s = jnp.log(jnp.clip(xg, 1e-30, None)) + g
    m = jnp.max(s, axis=1, keepdims=True)
    win = jnp.min(
        jnp.where(s == m, cols, jnp.int32(2**31 - 1)), axis=1, keepdims=True
    )
    return win
